# baseline (speedup 1.0000x reference)
import jax
import jax.numpy as jnp
from jax.experimental import pallas as pl
from jax.experimental.pallas import tpu as pltpu

G, N, F, H, B = 8, 1024, 128, 64, 16
GPB = 2


def _dot(p, q):
    return jnp.dot(p, q, preferred_element_type=jnp.float32)


def _gnn_body(xs_ref, a_hbm, win_ref, bin_ref, w0_ref, b0_ref, w1_ref,
              b1_ref, w2_ref, b2_ref, gids_ref, mask_ref, out_ref,
              abuf, sem):
    step = pl.program_id(0)
    nsteps = G // GPB

    @pl.when(step == 0)
    def _first():
        pltpu.make_async_copy(a_hbm.at[pl.ds(0, GPB)], abuf.at[0],
                              sem.at[0]).start()

    @pl.when(step + 1 < nsteps)
    def _prefetch():
        pltpu.make_async_copy(a_hbm.at[pl.ds((step + 1) * GPB, GPB)],
                              abuf.at[(step + 1) % 2],
                              sem.at[(step + 1) % 2]).start()

    pltpu.make_async_copy(a_hbm.at[pl.ds(step * GPB, GPB)],
                          abuf.at[step % 2], sem.at[step % 2]).wait()
    a = [abuf[step % 2, k] for k in range(GPB)]
    win, bin_ = win_ref[...], bin_ref[...]
    ws = [(w0_ref[...], b0_ref[...]), (w1_ref[...], b1_ref[...])]
    w2, b2 = w2_ref[...], b2_ref[...]

    x = [jnp.maximum(_dot(xs_ref[k], win) + bin_, 0.0) for k in range(GPB)]
    t = x
    for w, b in ws:
        y = [_dot(x[k], w) for k in range(GPB)]
        x = [jnp.maximum(_dot(a[k], y[k]) + b, 0.0) for k in range(GPB)]
    y = [_dot(x[k], w2) for k in range(GPB)]
    z = [_dot(a[k], y[k]) + b2 for k in range(GPB)]
    e = [jnp.exp(z[k]) for k in range(GPB)]
    x = [e[k] / jnp.sum(e[k], axis=-1, keepdims=True) + t[k]
         for k in range(GPB)]

    m = mask_ref[...]
    pm = [_dot(m, x[k]) for k in range(GPB)]
    cnt = jnp.maximum(jnp.sum(m, axis=1, keepdims=True), 1.0)

    @pl.when(step == 0)
    def _init():
        out_ref[...] = jnp.zeros_like(out_ref)

    acc = out_ref[...]
    for k in range(GPB):
        sel = gids_ref[...] == (step * GPB + k)
        acc = jnp.where(sel, pm[k] / cnt, acc)
    out_ref[...] = acc


@jax.jit
def kernel(cdfg_xs, cdfg_as, W_in, b_in, W0, b0, W1, b1, W2, b2, graph,
           coverpoint, coverpoint_mask):
    del coverpoint
    gids = graph.astype(jnp.int32).reshape(B, 1)
    maskf = coverpoint_mask.astype(jnp.float32)

    out = pl.pallas_call(
        _gnn_body,
        grid=(G // GPB,),
        in_specs=[
            pl.BlockSpec((GPB, N, F), lambda g: (g, 0, 0)),
            pl.BlockSpec(memory_space=pl.ANY),
            pl.BlockSpec((F, H), lambda g: (0, 0)),
            pl.BlockSpec((1, H), lambda g: (0, 0)),
            pl.BlockSpec((H, H), lambda g: (0, 0)),
            pl.BlockSpec((1, H), lambda g: (0, 0)),
            pl.BlockSpec((H, H), lambda g: (0, 0)),
            pl.BlockSpec((1, H), lambda g: (0, 0)),
            pl.BlockSpec((H, H), lambda g: (0, 0)),
            pl.BlockSpec((1, H), lambda g: (0, 0)),
            pl.BlockSpec((B, 1), lambda g: (0, 0)),
            pl.BlockSpec((B, N), lambda g: (0, 0)),
        ],
        out_specs=pl.BlockSpec((B, H), lambda g: (0, 0)),
        out_shape=jax.ShapeDtypeStruct((B, H), jnp.float32),
        scratch_shapes=[
            pltpu.VMEM((2, GPB, N, N), jnp.float32),
            pltpu.SemaphoreType.DMA((2,)),
        ],
    )(cdfg_xs, cdfg_as, W_in, b_in.reshape(1, H), W0, b0.reshape(1, H),
      W1, b1.reshape(1, H), W2, b2.reshape(1, H), gids, maskf)
    return out


# speedup vs baseline: 1.0409x; 1.0409x over previous
import jax
import jax.numpy as jnp
from jax.experimental import pallas as pl

G, N, F, H, B = 8, 1024, 128, 64, 16
GPB = 2


def _dot(p, q):
    return jnp.dot(p, q, preferred_element_type=jnp.float32)


def _gnn_body(xs_ref, a_ref, win_ref, bin_ref, w0_ref, b0_ref, w1_ref,
              b1_ref, w2_ref, b2_ref, gids_ref, mask_ref, out_ref):
    step = pl.program_id(0)
    a = [a_ref[k] for k in range(GPB)]
    win, bin_ = win_ref[...], bin_ref[...]
    ws = [(w0_ref[...], b0_ref[...]), (w1_ref[...], b1_ref[...])]
    w2, b2 = w2_ref[...], b2_ref[...]

    x = [jnp.maximum(_dot(xs_ref[k], win) + bin_, 0.0) for k in range(GPB)]
    t = x
    for w, b in ws:
        y = [_dot(x[k], w) for k in range(GPB)]
        x = [jnp.maximum(_dot(a[k], y[k]) + b, 0.0) for k in range(GPB)]
    y = [_dot(x[k], w2) for k in range(GPB)]
    z = [_dot(a[k], y[k]) + b2 for k in range(GPB)]
    e = [jnp.exp(z[k]) for k in range(GPB)]
    x = [e[k] / jnp.sum(e[k], axis=-1, keepdims=True) + t[k]
         for k in range(GPB)]

    m = mask_ref[...]
    pm = [_dot(m, x[k]) for k in range(GPB)]
    cnt = jnp.maximum(jnp.sum(m, axis=1, keepdims=True), 1.0)

    @pl.when(step == 0)
    def _init():
        out_ref[...] = jnp.zeros_like(out_ref)

    acc = out_ref[...]
    for k in range(GPB):
        sel = gids_ref[...] == (step * GPB + k)
        acc = jnp.where(sel, pm[k] / cnt, acc)
    out_ref[...] = acc


@jax.jit
def kernel(cdfg_xs, cdfg_as, W_in, b_in, W0, b0, W1, b1, W2, b2, graph,
           coverpoint, coverpoint_mask):
    del coverpoint
    gids = graph.astype(jnp.int32).reshape(B, 1)
    maskf = coverpoint_mask.astype(jnp.float32)

    out = pl.pallas_call(
        _gnn_body,
        grid=(G // GPB,),
        in_specs=[
            pl.BlockSpec((GPB, N, F), lambda g: (g, 0, 0)),
            pl.BlockSpec((GPB, N, N), lambda g: (g, 0, 0)),
            pl.BlockSpec((F, H), lambda g: (0, 0)),
            pl.BlockSpec((1, H), lambda g: (0, 0)),
            pl.BlockSpec((H, H), lambda g: (0, 0)),
            pl.BlockSpec((1, H), lambda g: (0, 0)),
            pl.BlockSpec((H, H), lambda g: (0, 0)),
            pl.BlockSpec((1, H), lambda g: (0, 0)),
            pl.BlockSpec((H, H), lambda g: (0, 0)),
            pl.BlockSpec((1, H), lambda g: (0, 0)),
            pl.BlockSpec((B, 1), lambda g: (0, 0)),
            pl.BlockSpec((B, N), lambda g: (0, 0)),
        ],
        out_specs=pl.BlockSpec((B, H), lambda g: (0, 0)),
        out_shape=jax.ShapeDtypeStruct((B, H), jnp.float32),
    )(cdfg_xs, cdfg_as, W_in, b_in.reshape(1, H), W0, b0.reshape(1, H),
      W1, b1.reshape(1, H), W2, b2.reshape(1, H), gids, maskf)
    return out
